# baseline (speedup 1.0000x reference)
import functools

import jax
import jax.numpy as jnp
from jax import lax
from jax.experimental import pallas as pl
from jax.experimental.pallas import tpu as pltpu
from jax.experimental.pallas import tpu_sc as plsc

N_NODES = 10000
N_EDGES = 320000
D = 128

NC = 2
NS = 16
L = 16
NW = NC * NS
K = 128
NCHUNK = 80
E_PAD = NW * NCHUNK * K
EPW = E_PAD // NW
N_PAD = 10240
RPT = N_PAD // NS
ZROWS = 64


def _mm_body(x_ref, w_ref, o_ref):
    o_ref[...] = jnp.dot(x_ref[...], w_ref[...],
                         preferred_element_type=jnp.float32)


def _matmul(x, W):
    return pl.pallas_call(
        _mm_body,
        grid=(10,),
        in_specs=[
            pl.BlockSpec((N_NODES // 10, D), lambda i: (i, 0)),
            pl.BlockSpec((D, D), lambda i: (0, 0)),
        ],
        out_specs=pl.BlockSpec((N_NODES // 10, D), lambda i: (i, 0)),
        out_shape=jax.ShapeDtypeStruct((N_NODES, D), jnp.float32),
    )(x, W)


def _merge_body(p_ref, o_ref):
    o_ref[...] = p_ref[0] + p_ref[1]


def _merge(partials):
    return pl.pallas_call(
        _merge_body,
        grid=(10,),
        in_specs=[pl.BlockSpec((NC, N_NODES // 10, D), lambda i: (0, i, 0))],
        out_specs=pl.BlockSpec((N_NODES // 10, D), lambda i: (i, 0)),
        out_shape=jax.ShapeDtypeStruct((N_NODES, D), jnp.float32),
    )(partials)


@functools.partial(
    pl.kernel,
    out_type=jax.ShapeDtypeStruct((NC, N_PAD, D), jnp.float32),
    mesh=plsc.VectorSubcoreMesh(core_axis_name="c", subcore_axis_name="s"),
    scratch_types=[
        pltpu.VMEM((K,), jnp.int32),
        pltpu.VMEM((K,), jnp.int32),
        pltpu.VMEM((K,), jnp.int32),
        pltpu.VMEM((K,), jnp.int32),
        pltpu.VMEM((K,), jnp.float32),
        pltpu.VMEM((K,), jnp.float32),
        pltpu.VMEM((K,), jnp.int32),
        pltpu.VMEM((K,), jnp.int32),
        pltpu.VMEM((K, D), jnp.float32),
        pltpu.VMEM((K, D), jnp.float32),
        pltpu.VMEM((ZROWS, D), jnp.float32),
        pltpu.VMEM_SHARED((N_PAD, D), jnp.float32),
        pltpu.SemaphoreType.DMA,
        pltpu.SemaphoreType.DMA,
        pltpu.SemaphoreType.DMA,
        pltpu.SemaphoreType.DMA,
        pltpu.SemaphoreType.DMA,
        pltpu.SemaphoreType.DMA,
    ],
)
def _aggregate(xw_hbm, src_hbm, dst_hbm, w_hbm, out_hbm,
               src0, src1, dst0, dst1, w0, w1, dsc0, dsc1,
               rows0, rows1, zbuf, acc,
               isem0, isem1, gsem0, gsem1, ssem0, ssem1):
    c = lax.axis_index("c")
    s = lax.axis_index("s")
    wid = s * NC + c
    srcb = (src0, src1)
    dstb = (dst0, dst1)
    wb = (w0, w1)
    dsc = (dsc0, dsc1)
    rows = (rows0, rows1)
    isems = (isem0, isem1)
    gsems = (gsem0, gsem1)
    ssems = (ssem0, ssem1)
    last = NCHUNK - 1

    def _prefetch(ci, b):
        ci = jnp.minimum(ci, last)
        pltpu.async_copy(src_hbm.at[wid, ci], srcb[b], isems[b])
        pltpu.async_copy(dst_hbm.at[wid, ci], dstb[b], isems[b])
        pltpu.async_copy(w_hbm.at[wid, ci], wb[b], isems[b])

    def _wait_prefetch(b):
        pltpu.make_async_copy(src_hbm.at[wid, 0], srcb[b], isems[b]).wait()
        pltpu.make_async_copy(dst_hbm.at[wid, 0], dstb[b], isems[b]).wait()
        pltpu.make_async_copy(w_hbm.at[wid, 0], wb[b], isems[b]).wait()

    def _gather(b):
        pltpu.async_copy(xw_hbm.at[srcb[b]], rows[b], gsems[b])

    def _wait_gather(b):
        pltpu.make_async_copy(xw_hbm.at[srcb[b]], rows[b], gsems[b]).wait()

    _prefetch(0, 0)
    _wait_prefetch(0)
    _gather(0)
    _prefetch(1, 1)

    def _zero_buf(i, carry):
        for g in range(D // L):
            zbuf[i, pl.ds(g * L, L)] = jnp.zeros((L,), jnp.float32)
        return carry

    lax.fori_loop(0, ZROWS, _zero_buf, 0)

    def _zero_acc(j, carry):
        pltpu.async_copy(zbuf, acc.at[pl.ds(s * RPT + j * ZROWS, ZROWS)],
                         gsems[1])
        return carry

    lax.fori_loop(0, RPT // ZROWS, _zero_acc, 0)

    def _zero_drain(j, carry):
        pltpu.make_async_copy(
            zbuf, acc.at[pl.ds(s * RPT, ZROWS)], gsems[1]).wait()
        return carry

    lax.fori_loop(0, RPT // ZROWS, _zero_drain, 0)
    plsc.subcore_barrier()

    def _scale(b):
        def _body(eg, inner):
            e0 = eg * L
            wv = wb[b][pl.ds(e0, L)]
            for j in range(L):
                we = lax.gather(
                    wv, jnp.full((L, 1), j, jnp.int32),
                    lax.GatherDimensionNumbers(
                        offset_dims=(), collapsed_slice_dims=(0,),
                        start_index_map=(0,)),
                    slice_sizes=(1,),
                    mode=lax.GatherScatterMode.PROMISE_IN_BOUNDS)
                for g in range(D // L):
                    rows[b][e0 + j, pl.ds(g * L, L)] = (
                        rows[b][e0 + j, pl.ds(g * L, L)] * we)
            return inner

        lax.fori_loop(0, K // L, _body, 0)

    def _scatter(b):
        for g in range(K // L):
            dsc[b][pl.ds(g * L, L)] = dstb[b][pl.ds(g * L, L)]
        pltpu.async_copy(rows[b], acc.at[dsc[b]], ssems[b], add=True)

    def _wait_scatter(b):
        pltpu.make_async_copy(rows[b], acc.at[dsc[b]], ssems[b]).wait()

    def _step(ci, b, first):
        _wait_gather(b)
        _wait_prefetch(b ^ 1)
        if not first:
            _wait_scatter(b ^ 1)
        _gather(b ^ 1)
        _scale(b)
        _scatter(b)
        _prefetch(ci + 2, b)

    _step(0, 0, True)
    _step(1, 1, False)

    def _pair(i, carry):
        c0 = 2 + i * 2
        _step(c0, 0, False)
        _step(c0 + 1, 1, False)
        return carry

    lax.fori_loop(0, (NCHUNK - 2) // 2, _pair, 0)
    _wait_gather(0)
    _wait_prefetch(1)
    _wait_scatter(1)

    plsc.subcore_barrier()
    pltpu.sync_copy(acc.at[pl.ds(s * RPT, RPT)],
                    out_hbm.at[c, pl.ds(s * RPT, RPT)])


def kernel(x, edge_index, edge_weight, W):
    xw = _matmul(x, W)
    pad = E_PAD - N_EDGES
    src = jnp.pad(edge_index[0].astype(jnp.int32), (0, pad))
    dst = jnp.pad(edge_index[1].astype(jnp.int32), (0, pad))
    ew = jnp.pad(edge_weight, (0, pad))
    partials = _aggregate(xw, src.reshape(NW, NCHUNK, K),
                          dst.reshape(NW, NCHUNK, K),
                          ew.reshape(NW, NCHUNK, K))
    return _merge(partials)


# speedup vs baseline: 2.9985x; 2.9985x over previous
import functools

import jax
import jax.numpy as jnp
from jax import lax
from jax.experimental import pallas as pl
from jax.experimental.pallas import tpu as pltpu
from jax.experimental.pallas import tpu_sc as plsc

N_NODES = 10000
N_EDGES = 320000
D = 128

NC = 2
NS = 16
L = 16
NW = NC * NS
K = 128
NCHUNK = 80
E_PAD = NW * NCHUNK * K
EPW = E_PAD // NW
N_PAD = 10240
RPT = N_PAD // NS
ZROWS = 64


def _mm_body(x_ref, w_ref, o_ref):
    o_ref[...] = jnp.dot(x_ref[...], w_ref[...],
                         preferred_element_type=jnp.float32)


def _matmul(x, W):
    return pl.pallas_call(
        _mm_body,
        grid=(10,),
        in_specs=[
            pl.BlockSpec((N_NODES // 10, D), lambda i: (i, 0)),
            pl.BlockSpec((D, D), lambda i: (0, 0)),
        ],
        out_specs=pl.BlockSpec((N_NODES // 10, D), lambda i: (i, 0)),
        out_shape=jax.ShapeDtypeStruct((N_NODES, D), jnp.float32),
    )(x, W)


def _merge_body(p_ref, o_ref):
    o_ref[...] = p_ref[0] + p_ref[1]


def _merge(partials):
    return pl.pallas_call(
        _merge_body,
        grid=(10,),
        in_specs=[pl.BlockSpec((NC, N_NODES // 10, D), lambda i: (0, i, 0))],
        out_specs=pl.BlockSpec((N_NODES // 10, D), lambda i: (i, 0)),
        out_shape=jax.ShapeDtypeStruct((N_NODES, D), jnp.float32),
    )(partials)


@functools.partial(
    pl.kernel,
    out_type=jax.ShapeDtypeStruct((NC, N_PAD, D), jnp.float32),
    mesh=plsc.VectorSubcoreMesh(core_axis_name="c", subcore_axis_name="s"),
    scratch_types=[
        pltpu.VMEM((K,), jnp.int32),
        pltpu.VMEM((K,), jnp.int32),
        pltpu.VMEM((K,), jnp.int32),
        pltpu.VMEM((K,), jnp.int32),
        pltpu.VMEM((K,), jnp.float32),
        pltpu.VMEM((K,), jnp.float32),
        pltpu.VMEM((K,), jnp.int32),
        pltpu.VMEM((K,), jnp.int32),
        pltpu.VMEM((K, D), jnp.float32),
        pltpu.VMEM((K, D), jnp.float32),
        pltpu.VMEM((ZROWS, D), jnp.float32),
        pltpu.VMEM_SHARED((N_PAD, D), jnp.float32),
        pltpu.SemaphoreType.DMA,
        pltpu.SemaphoreType.DMA,
        pltpu.SemaphoreType.DMA,
        pltpu.SemaphoreType.DMA,
        pltpu.SemaphoreType.DMA,
        pltpu.SemaphoreType.DMA,
    ],
)
def _aggregate(xw_hbm, src_hbm, dst_hbm, w_hbm, out_hbm,
               src0, src1, dst0, dst1, w0, w1, dsc0, dsc1,
               rows0, rows1, zbuf, acc,
               isem0, isem1, gsem0, gsem1, ssem0, ssem1):
    c = lax.axis_index("c")
    s = lax.axis_index("s")
    wid = s * NC + c
    srcb = (src0, src1)
    dstb = (dst0, dst1)
    wb = (w0, w1)
    dsc = (dsc0, dsc1)
    rows = (rows0, rows1)
    isems = (isem0, isem1)
    gsems = (gsem0, gsem1)
    ssems = (ssem0, ssem1)
    last = NCHUNK - 1

    def _prefetch(ci, b):
        ci = jnp.minimum(ci, last)
        pltpu.async_copy(src_hbm.at[wid, ci], srcb[b], isems[b])
        pltpu.async_copy(dst_hbm.at[wid, ci], dstb[b], isems[b])
        pltpu.async_copy(w_hbm.at[wid, ci], wb[b], isems[b])

    def _wait_prefetch(b):
        pltpu.make_async_copy(src_hbm.at[wid, 0], srcb[b], isems[b]).wait()
        pltpu.make_async_copy(dst_hbm.at[wid, 0], dstb[b], isems[b]).wait()
        pltpu.make_async_copy(w_hbm.at[wid, 0], wb[b], isems[b]).wait()

    def _gather(b):
        pltpu.async_copy(xw_hbm.at[srcb[b]], rows[b], gsems[b])

    def _wait_gather(b):
        pltpu.make_async_copy(xw_hbm.at[srcb[b]], rows[b], gsems[b]).wait()

    _prefetch(0, 0)
    _wait_prefetch(0)
    _gather(0)
    _prefetch(1, 1)

    def _zero_buf(i, carry):
        for g in range(D // L):
            zbuf[i, pl.ds(g * L, L)] = jnp.zeros((L,), jnp.float32)
        return carry

    lax.fori_loop(0, ZROWS, _zero_buf, 0)

    def _zero_acc(j, carry):
        pltpu.async_copy(zbuf, acc.at[pl.ds(s * RPT + j * ZROWS, ZROWS)],
                         gsems[1])
        return carry

    lax.fori_loop(0, RPT // ZROWS, _zero_acc, 0)

    def _zero_drain(j, carry):
        pltpu.make_async_copy(
            zbuf, acc.at[pl.ds(s * RPT, ZROWS)], gsems[1]).wait()
        return carry

    lax.fori_loop(0, RPT // ZROWS, _zero_drain, 0)
    plsc.subcore_barrier()

    def _scale(b):
        def _body(eg, inner):
            e0 = eg * L
            wv = wb[b][pl.ds(e0, L)]
            for j in range(L):
                we = lax.gather(
                    wv, jnp.full((L, 1), j, jnp.int32),
                    lax.GatherDimensionNumbers(
                        offset_dims=(), collapsed_slice_dims=(0,),
                        start_index_map=(0,)),
                    slice_sizes=(1,),
                    mode=lax.GatherScatterMode.PROMISE_IN_BOUNDS)
                for g in range(D // L):
                    rows[b][e0 + j, pl.ds(g * L, L)] = (
                        rows[b][e0 + j, pl.ds(g * L, L)] * we)
            return inner

        lax.fori_loop(0, K // L, _body, 0)

    def _scatter(b):
        for g in range(K // L):
            dsc[b][pl.ds(g * L, L)] = dstb[b][pl.ds(g * L, L)]
        pltpu.async_copy(rows[b], acc.at[dsc[b]], ssems[b], add=True)

    def _wait_scatter(b):
        pltpu.make_async_copy(rows[b], acc.at[dsc[b]], ssems[b]).wait()

    def _step(ci, b, first):
        _wait_gather(b)
        _wait_prefetch(b ^ 1)
        if not first:
            _wait_scatter(b ^ 1)
        _gather(b ^ 1)
        _scale(b)
        _scatter(b)
        _prefetch(ci + 2, b)

    _step(0, 0, True)
    _step(1, 1, False)

    def _pair(i, carry):
        c0 = 2 + i * 2
        _step(c0, 0, False)
        _step(c0 + 1, 1, False)
        return carry

    lax.fori_loop(0, (NCHUNK - 2) // 2, _pair, 0)
    _wait_gather(0)
    _wait_prefetch(1)
    _wait_scatter(1)

    plsc.subcore_barrier()
    pltpu.sync_copy(acc.at[pl.ds(s * RPT, RPT)],
                    out_hbm.at[c, pl.ds(s * RPT, RPT)])


def kernel(x, edge_index, edge_weight, W):
    xw = _matmul(x, W)
    pad = E_PAD - N_EDGES
    spread = jnp.arange(pad, dtype=jnp.int32) % N_NODES
    src = jnp.concatenate([edge_index[0].astype(jnp.int32), spread])
    dst = jnp.concatenate([edge_index[1].astype(jnp.int32), spread])
    ew = jnp.pad(edge_weight, (0, pad))
    partials = _aggregate(xw, src.reshape(NW, NCHUNK, K),
                          dst.reshape(NW, NCHUNK, K),
                          ew.reshape(NW, NCHUNK, K))
    return _merge(partials)
